# Initial kernel scaffold; baseline (speedup 1.0000x reference)
import jax
import jax.numpy as jnp
from jax.experimental import pallas as pl


def kernel(x, embed, W_ih, W_hh, b_ih, b_hh, W_out, b_out):
    raise NotImplementedError("write your pallas kernel here")


# speedup vs baseline: 8.9414x; 8.9414x over previous
import jax
import jax.numpy as jnp
from jax import lax
from jax.experimental import pallas as pl
from jax.experimental.pallas import tpu as pltpu

VOCAB = 100000
VP = 102400
CW = 12800
NCH = VP // CW
H = 128
T = 20
K = 50
NEG = -1e30
BIGI = 2**30


def _step(x_smem, embed_hbm, wout, bout2, wih_t, whh_t, bias2,
          probs_out, ah_out, fb_out, hs_out, cs_out,
          lscr, code8, h8_ref, c8_ref, emb_v, ahat_s, fbf_s, dma_sem):
    t = pl.program_id(0)

    @pl.when(t == 0)
    def _init():
        h8_ref[...] = jnp.zeros((8, H), jnp.float32)
        c8_ref[...] = jnp.zeros((8, H), jnp.float32)

        def init_body(c, _):
            base = pl.multiple_of(c * CW, 512)
            iota = lax.broadcasted_iota(jnp.int32, (8, CW), 1) + base
            cand = jnp.zeros((8, CW), jnp.bool_)
            for j in range(T):
                cand = cand | (iota == x_smem[j] - 1)
            codef = jnp.where(cand, 3.0, 1.0)
            code8[:, pl.ds(base, CW)] = codef.astype(jnp.bfloat16)
            return 0

        lax.fori_loop(0, NCH, init_body, 0)
        emb_v[...] = jnp.zeros((1, H), jnp.float32)
        fbf_s[0] = 1.0
        ahat_s[0] = 0

    @pl.when(t > 0)
    def _fetch_emb():
        idx = ahat_s[0]
        cp = pltpu.make_async_copy(embed_hbm.at[pl.ds(idx, 1)], emb_v, dma_sem)
        cp.start()
        cp.wait()

    emb8 = jnp.broadcast_to(emb_v[...], (8, H)) * fbf_s[0]
    h8 = h8_ref[...]
    gates = (
        lax.dot_general(emb8, wih_t[...], (((1,), (0,)), ((), ())),
                        preferred_element_type=jnp.float32)
        + lax.dot_general(h8, whh_t[...], (((1,), (0,)), ((), ())),
                          preferred_element_type=jnp.float32)
        + bias2[...]
    )
    ig = gates[:, 0:H]
    fg = gates[:, H:2 * H]
    gg = gates[:, 2 * H:3 * H]
    og = gates[:, 3 * H:4 * H]
    c8 = jax.nn.sigmoid(fg) * c8_ref[...] + jax.nn.sigmoid(ig) * jnp.tanh(gg)
    h8n = jax.nn.sigmoid(og) * jnp.tanh(c8)
    c8_ref[...] = c8
    h8_ref[...] = h8n

    def p1_body(c, carry):
        m, mx, vb = carry
        base = pl.multiple_of(c * CW, 512)
        l = lax.dot_general(h8n, wout[pl.ds(base, CW), :],
                            (((1,), (1,)), ((), ())),
                            preferred_element_type=jnp.float32) \
            + bout2[:, pl.ds(base, CW)]
        lscr[:, pl.ds(base, CW)] = l
        code = code8[:, pl.ds(base, CW)].astype(jnp.float32)
        s = jnp.where((code == 1.0) | (code == 3.0), l, NEG)
        m = jnp.maximum(m, jnp.max(l))
        mx = jnp.maximum(mx, jnp.max(s))
        vb = jnp.maximum(vb, jnp.max(jnp.where(code >= 2.0, s, NEG)))
        return m, mx, vb

    m, mx, vb = lax.fori_loop(
        0, NCH, p1_body,
        (jnp.float32(NEG), jnp.float32(NEG), jnp.float32(NEG)))

    def p2_body(c, carry):
        zsum, gidx, cidx, cnt_g = carry
        base = pl.multiple_of(c * CW, 512)
        iota = lax.broadcasted_iota(jnp.int32, (8, CW), 1) + base
        l = lscr[:, pl.ds(base, CW)]
        code = code8[:, pl.ds(base, CW)].astype(jnp.float32)
        maskb = (code == 1.0) | (code == 3.0)
        s = jnp.where(maskb, l, NEG)
        zsum = zsum + jnp.sum(jnp.exp(l - m))
        cnt_g = cnt_g + jnp.sum((s > vb).astype(jnp.int32))
        gidx = jnp.minimum(gidx, jnp.min(jnp.where(s == mx, iota, BIGI)))
        cs = jnp.where(code >= 2.0, s, NEG)
        cidx = jnp.minimum(cidx, jnp.min(jnp.where(cs == vb, iota, BIGI)))
        return zsum, gidx, cidx, cnt_g

    zsum, gidx, cidx, cnt_g = lax.fori_loop(
        0, NCH, p2_body,
        (jnp.float32(0.0), jnp.int32(BIGI), jnp.int32(BIGI), jnp.int32(0)))
    zinv = 8.0 / zsum
    cnt_g = cnt_g // 8

    def p3_body(c, cnt_e):
        base = pl.multiple_of(c * CW, 512)
        iota = lax.broadcasted_iota(jnp.int32, (8, CW), 1) + base
        l = lscr[:, pl.ds(base, CW)]
        code = code8[:, pl.ds(base, CW)].astype(jnp.float32)
        maskb = (code == 1.0) | (code == 3.0)
        s = jnp.where(maskb, l, NEG)
        p = jnp.exp(l - m) * zinv * jnp.where(maskb, 1.0, 0.0)
        probs_out[0, :, pl.ds(base, CW)] = p[0:1, :]
        cnt_e = cnt_e + jnp.sum(((s == vb) & (iota < cidx)).astype(jnp.int32))
        return cnt_e

    cnt_e = lax.fori_loop(0, NCH, p3_body, jnp.int32(0)) // 8

    hit = (cnt_g + cnt_e) <= K - 1
    a_hat = jnp.where(hit, cidx, gidx).astype(jnp.int32)
    fb = jnp.where(hit, jnp.int32(1), jnp.int32(-1))

    cbase = pl.multiple_of((a_hat // CW) * CW, 512)
    off = a_hat % CW
    iota = lax.broadcasted_iota(jnp.int32, (8, CW), 1)
    code = code8[:, pl.ds(cbase, CW)].astype(jnp.float32)
    code8[:, pl.ds(cbase, CW)] = jnp.where(
        iota == off, code - 1.0, code).astype(jnp.bfloat16)

    ahat_s[0] = a_hat
    fbf_s[0] = fb.astype(jnp.float32)

    ah_out[...] = jnp.full((1, 1, 128), a_hat, jnp.int32)
    fb_out[...] = jnp.full((1, 1, 128), fb, jnp.int32)
    hs_out[...] = h8n[0:1, :].reshape(1, 1, H)
    cs_out[...] = c8[0:1, :].reshape(1, 1, H)


@jax.jit
def _run(x, embed, W_ih, W_hh, b_ih, b_hh, W_out, b_out):
    xi = x.astype(jnp.int32)
    wout_p = jnp.pad(W_out, ((0, VP - VOCAB), (0, 0)))
    bout_p = jnp.pad(b_out, (0, VP - VOCAB), constant_values=NEG).reshape(1, VP)
    wih_t = W_ih.T
    whh_t = W_hh.T
    bias2 = (b_ih + b_hh).reshape(1, 4 * H)

    probs3, ah3, fb3, hs3, cs3 = pl.pallas_call(
        _step,
        grid=(T,),
        in_specs=[
            pl.BlockSpec(memory_space=pltpu.SMEM),
            pl.BlockSpec(memory_space=pl.ANY),
            pl.BlockSpec((VP, H), lambda t: (0, 0)),
            pl.BlockSpec((1, VP), lambda t: (0, 0)),
            pl.BlockSpec((H, 4 * H), lambda t: (0, 0)),
            pl.BlockSpec((H, 4 * H), lambda t: (0, 0)),
            pl.BlockSpec((1, 4 * H), lambda t: (0, 0)),
        ],
        out_specs=[
            pl.BlockSpec((1, 1, VP), lambda t: (t, 0, 0)),
            pl.BlockSpec((1, 1, 128), lambda t: (t, 0, 0)),
            pl.BlockSpec((1, 1, 128), lambda t: (t, 0, 0)),
            pl.BlockSpec((1, 1, H), lambda t: (t, 0, 0)),
            pl.BlockSpec((1, 1, H), lambda t: (t, 0, 0)),
        ],
        out_shape=[
            jax.ShapeDtypeStruct((T, 1, VP), jnp.float32),
            jax.ShapeDtypeStruct((T, 1, 128), jnp.int32),
            jax.ShapeDtypeStruct((T, 1, 128), jnp.int32),
            jax.ShapeDtypeStruct((T, 1, H), jnp.float32),
            jax.ShapeDtypeStruct((T, 1, H), jnp.float32),
        ],
        scratch_shapes=[
            pltpu.VMEM((8, VP), jnp.float32),
            pltpu.VMEM((8, VP), jnp.bfloat16),
            pltpu.VMEM((8, H), jnp.float32),
            pltpu.VMEM((8, H), jnp.float32),
            pltpu.VMEM((1, H), jnp.float32),
            pltpu.SMEM((1,), jnp.int32),
            pltpu.SMEM((1,), jnp.float32),
            pltpu.SemaphoreType.DMA,
        ],
        compiler_params=pltpu.CompilerParams(
            dimension_semantics=("arbitrary",),
        ),
    )(xi, embed, wout_p, bout_p, wih_t, whh_t, bias2)

    a_hats = ah3[:, 0, 0]
    feedbacks = fb3[:, 0, 0]
    probs = probs3.reshape(T, VP)[:, :VOCAB]
    hs = hs3[:, 0, :]
    cs = cs3[:, 0, :]
    return a_hats, feedbacks, probs, (hs, cs)


def kernel(x, embed, W_ih, W_hh, b_ih, b_hh, W_out, b_out):
    return _run(x, embed, W_ih, W_hh, b_ih, b_hh, W_out, b_out)
